# Initial kernel scaffold; baseline (speedup 1.0000x reference)
import jax
import jax.numpy as jnp
from jax.experimental import pallas as pl


def kernel(z_e_x, c, emb_weight):
    raise NotImplementedError("write your pallas kernel here")


# speedup vs baseline: 3.3115x; 3.3115x over previous
import jax
import jax.numpy as jnp
from jax.experimental import pallas as pl
from jax.experimental.pallas import tpu as pltpu

_K = 512
_D = 64
_NUM_CLASSES = 60
_N = 1024
_NB = 256


def _vq_body(c_ref, z_ref, cb_ref, out_ref):
    z = z_ref[0]
    sub = cb_ref[0]
    e_sq = jnp.sum(sub * sub, axis=1, keepdims=True)
    cross = jax.lax.dot_general(
        sub, z, (((1,), (0,)), ((), ())),
        preferred_element_type=jnp.float32)
    dist = e_sq - 2.0 * cross
    minv = jnp.min(dist, axis=0, keepdims=True)
    iota = jax.lax.broadcasted_iota(jnp.int32, (_K, _NB), 0)
    idx = jnp.min(jnp.where(dist == minv, iota, _K), axis=0, keepdims=True)
    onehot = (iota == idx).astype(jnp.float32)
    quant = jax.lax.dot_general(
        onehot, sub, (((0,), (0,)), ((), ())),
        preferred_element_type=jnp.float32)
    out_ref[0] = quant


def kernel(z_e_x, c, emb_weight):
    B = z_e_x.shape[0]
    z2 = z_e_x.reshape(B, _D, _N)
    codebooks = emb_weight.reshape(_NUM_CLASSES, _K, _D)
    grid_spec = pltpu.PrefetchScalarGridSpec(
        num_scalar_prefetch=1,
        grid=(B, _N // _NB),
        in_specs=[
            pl.BlockSpec((1, _D, _NB), lambda b, n, c_ref: (b, 0, n)),
            pl.BlockSpec((1, _K, _D), lambda b, n, c_ref: (c_ref[b], 0, 0)),
        ],
        out_specs=pl.BlockSpec((1, _NB, _D), lambda b, n, c_ref: (b, n, 0)),
    )
    out = pl.pallas_call(
        _vq_body,
        grid_spec=grid_spec,
        out_shape=jax.ShapeDtypeStruct((B, _N, _D), jnp.float32),
    )(c, z2, codebooks)
    return out.reshape(B, 32, 32, _D)
